# baseline (speedup 1.0000x reference)
import functools

import jax
import jax.numpy as jnp
from jax import lax
from jax.experimental import pallas as pl
from jax.experimental.pallas import tpu as pltpu
from jax.experimental.pallas import tpu_sc as plsc

VOCAB = 100000
LATENT = 32
NUM_DISC = 26
NUM_CONT = 16
HID = 64

_NC = 2
_NS = 16
_NW = _NC * _NS

_SLOTS = 28
_PLANES = 7
_GATHER_ROWS = 128
_CTOK = 256
_GPP = _CTOK * 4 // _GATHER_ROWS
_L = 16


def _sc_gather(table, cate2d, incre2d, n_tokens):
    rows_total = _SLOTS * n_tokens
    tok_per_w = n_tokens // _NW
    chunks = tok_per_w // _CTOK
    crows = _CTOK * 4 // _L
    mesh = plsc.VectorSubcoreMesh(core_axis_name="c", subcore_axis_name="s")

    @functools.partial(
        pl.kernel,
        mesh=mesh,
        out_type=jax.ShapeDtypeStruct((rows_total, 32), jnp.float32),
        scratch_types=[
            pltpu.VMEM((_CTOK * 20 // 128, 128), jnp.int32),
            pltpu.VMEM((_CTOK * 6 // 128, 128), jnp.int32),
            pltpu.VMEM((_PLANES * _GPP, 128), jnp.int32),
            pltpu.VMEM((_CTOK * 4, 32), jnp.float32),
            pltpu.SemaphoreType.DMA,
        ],
        compiler_params=pltpu.CompilerParams(use_tc_tiling_on_sc=False,
                                             needs_layout_passes=False),
    )
    def k(table_hbm, cate_hbm, incre_hbm, out_hbm, cate_v, incre_v, idx_p,
          rows_v, sem):
        wid = lax.axis_index("s") * _NC + lax.axis_index("c")
        tok0_w = wid * tok_per_w

        iota = lax.iota(jnp.int32, _L)
        tq = iota >> 2
        sj = iota & 3
        pat_cate = tq * 20 + sj
        pat_inc5 = tq * 6 + sj
        pat_inc6 = tq * 6 + 4 + (iota & 1)

        def body(i, _):
            tok0 = tok0_w + i * _CTOK
            pltpu.sync_copy(cate_hbm.at[pl.ds(tok0 * 20 // 128,
                                              _CTOK * 20 // 128)], cate_v)
            pltpu.sync_copy(incre_hbm.at[pl.ds(tok0 * 6 // 128,
                                               _CTOK * 6 // 128)], incre_v)
            for plane in range(_PLANES):
                if plane < 5:
                    pat, src, stride = pat_cate + 4 * plane, cate_v, 80
                elif plane == 5:
                    pat, src, stride = pat_inc5, incre_v, 24
                else:
                    pat, src, stride = pat_inc6, incre_v, 24
                for v in range(crows):
                    off = pat + stride * v
                    vals = plsc.load_gather(src, [off >> 7, off & 127])
                    idx_p[plane * _GPP + v // 8,
                          pl.ds(_L * (v % 8), _L)] = vals

            for plane in range(_PLANES):
                descs = [
                    pltpu.async_copy(
                        table_hbm.at[idx_p.at[plane * _GPP + j]],
                        rows_v.at[pl.ds(j * _GATHER_ROWS, _GATHER_ROWS)],
                        sem,
                    )
                    for j in range(_GPP)
                ]
                for d in descs:
                    d.wait()
                pltpu.sync_copy(
                    rows_v,
                    out_hbm.at[pl.ds(plane * n_tokens * 4 + tok0 * 4,
                                     _CTOK * 4)])
            return _

        lax.fori_loop(0, chunks, body, None)

    return k(table, cate2d, incre2d)


def _tc_head(g3, aux, Wc, Waux, W2, b1, b2, fct, fcb):
    N = g3.shape[1]
    T = 1024
    grid = (N // T,)

    def body(g_ref, aux_ref, wc_ref, waux_ref, w2_ref, b1_ref, b2_ref,
             fct_ref, fcb_ref, rate_ref, tmu_ref, tsd_ref):
        u = jnp.dot(aux_ref[...], waux_ref[...],
                    preferred_element_type=jnp.float32)
        for k in range(_PLANES):
            u = u + jnp.dot(g_ref[k], wc_ref[pl.ds(128 * k, 128), :],
                            preferred_element_type=jnp.float32)
        h = jnp.tanh(u[:, :HID] + b1_ref[...])
        t_out = jnp.dot(h, w2_ref[...], preferred_element_type=jnp.float32)
        t_out = t_out + b2_ref[...]
        t_mu = t_out[:, :LATENT]
        tmu_ref[...] = t_mu
        tsd_ref[...] = jnp.exp(0.5 * t_out[:, LATENT:])
        v = (u[:, HID:HID + 1]
             + jnp.sum(t_mu * fct_ref[...], axis=1, keepdims=True)
             + fcb_ref[...])
        rate_ref[...] = jnp.exp(v)

    return pl.pallas_call(
        body,
        grid=grid,
        in_specs=[
            pl.BlockSpec((_PLANES, T, 128), lambda i: (0, i, 0)),
            pl.BlockSpec((T, 18), lambda i: (i, 0)),
            pl.BlockSpec((_PLANES * 128, 65), lambda i: (0, 0)),
            pl.BlockSpec((18, 65), lambda i: (0, 0)),
            pl.BlockSpec((HID, HID), lambda i: (0, 0)),
            pl.BlockSpec((1, HID), lambda i: (0, 0)),
            pl.BlockSpec((1, 2 * LATENT), lambda i: (0, 0)),
            pl.BlockSpec((1, LATENT), lambda i: (0, 0)),
            pl.BlockSpec((1, 1), lambda i: (0, 0)),
        ],
        out_specs=[
            pl.BlockSpec((T, 1), lambda i: (i, 0)),
            pl.BlockSpec((T, LATENT), lambda i: (i, 0)),
            pl.BlockSpec((T, LATENT), lambda i: (i, 0)),
        ],
        out_shape=[
            jax.ShapeDtypeStruct((N, 1), jnp.float32),
            jax.ShapeDtypeStruct((N, LATENT), jnp.float32),
            jax.ShapeDtypeStruct((N, LATENT), jnp.float32),
        ],
        compiler_params=pltpu.CompilerParams(
            dimension_semantics=("parallel",),
        ),
    )(g3, aux, Wc, Waux, W2, b1, b2, fct, fcb)


def kernel(cate, cat_incre, obs, target, time, emb_mu, emb_logvar, itemw_mu,
           t_W1, t_b1, t_W2, t_b2, fc_W, fc_b):
    B, S = cate.shape[0], cate.shape[1]
    N = B * S
    EMB = NUM_DISC * LATENT

    Wc = jnp.concatenate(
        [jnp.concatenate([t_W1[:EMB], fc_W[:EMB]], axis=1),
         jnp.zeros((_SLOTS * LATENT - EMB, HID + 1), jnp.float32)],
        axis=0)
    Waux = jnp.concatenate(
        [t_W1[EMB:EMB + NUM_CONT + 2],
         jnp.concatenate([fc_W[EMB + LATENT:], jnp.zeros((2, 1), jnp.float32)],
                         axis=0)],
        axis=1)
    fct = fc_W[EMB:EMB + LATENT].reshape(1, LATENT)

    halves = []
    HB = B // 2
    HN = HB * S
    for h in range(2):
        cate_h = cate[h * HB:(h + 1) * HB]
        incre_h = cat_incre[h * HB:(h + 1) * HB]
        cate2d = cate_h.astype(jnp.int32).reshape(HN * 20 // 128, 128)
        incre2d = incre_h.astype(jnp.int32).reshape(HN * 6 // 128, 128)
        g3 = _sc_gather(emb_mu, cate2d, incre2d, HN).reshape(_PLANES, HN, 128)
        aux = jnp.concatenate(
            [obs[h * HB:(h + 1) * HB].reshape(HN, NUM_CONT),
             target[h * HB:(h + 1) * HB].reshape(HN, 1),
             time[h * HB:(h + 1) * HB].reshape(HN, 1)], axis=1)
        halves.append(_tc_head(
            g3, aux, Wc, Waux, t_W2,
            t_b1.reshape(1, HID), t_b2.reshape(1, 2 * LATENT),
            fct, fc_b.reshape(1, 1)))

    rate = jnp.concatenate(
        [halves[0][0].reshape(HB, S), halves[1][0].reshape(HB, S)], axis=0)
    t_mu = jnp.concatenate(
        [halves[0][1].reshape(HB, S, LATENT),
         halves[1][1].reshape(HB, S, LATENT)], axis=0)
    t_sd = jnp.concatenate(
        [halves[0][2].reshape(HB, S, LATENT),
         halves[1][2].reshape(HB, S, LATENT)], axis=0)

    q_item = (emb_mu, jnp.exp(0.5 * emb_logvar))
    q_itemw = (itemw_mu, jnp.ones_like(itemw_mu))
    q_time = (t_mu, t_sd)
    p_item = (jnp.zeros_like(emb_mu), jnp.ones_like(emb_mu))
    p_itemw = (jnp.zeros_like(itemw_mu), jnp.ones_like(itemw_mu))
    p_time = (jnp.zeros_like(t_mu), jnp.ones_like(t_mu))
    return (rate, q_item, q_itemw, q_time, p_item, p_itemw, p_time)


# speedup vs baseline: 1.1045x; 1.1045x over previous
import functools

import jax
import jax.numpy as jnp
from jax import lax
from jax.experimental import pallas as pl
from jax.experimental.pallas import tpu as pltpu
from jax.experimental.pallas import tpu_sc as plsc

VOCAB = 100000
LATENT = 32
NUM_DISC = 26
NUM_CONT = 16
HID = 64

_NC = 2
_NS = 16
_NW = _NC * _NS

_SLOTS = 28
_PLANES = 7
_GATHER_ROWS = 128
_CTOK = 256
_GPP = _CTOK * 4 // _GATHER_ROWS
_L = 16


def _sc_gather(table, cate2d, incre2d, n_tokens):
    rows_total = _SLOTS * n_tokens
    tok_per_w = n_tokens // _NW
    chunks = tok_per_w // _CTOK
    crows = _CTOK * 4 // _L
    mesh = plsc.VectorSubcoreMesh(core_axis_name="c", subcore_axis_name="s")

    @functools.partial(
        pl.kernel,
        mesh=mesh,
        out_type=jax.ShapeDtypeStruct((rows_total, 32), jnp.float32),
        scratch_types=[
            pltpu.VMEM((_CTOK * 20 // 128, 128), jnp.int32),
            pltpu.VMEM((_CTOK * 6 // 128, 128), jnp.int32),
            pltpu.VMEM((_PLANES * _GPP, 128), jnp.int32),
            pltpu.VMEM((2, _CTOK * 4, 32), jnp.float32),
            pltpu.SemaphoreType.DMA,
            pltpu.SemaphoreType.DMA,
        ],
        compiler_params=pltpu.CompilerParams(use_tc_tiling_on_sc=False,
                                             needs_layout_passes=False),
    )
    def k(table_hbm, cate_hbm, incre_hbm, out_hbm, cate_v, incre_v, idx_p,
          rows_v, sem, wsem):
        wid = lax.axis_index("s") * _NC + lax.axis_index("c")
        tok0_w = wid * tok_per_w

        iota = lax.iota(jnp.int32, _L)
        tq = iota >> 2
        sj = iota & 3
        pat_cate = tq * 20 + sj
        pat_inc5 = tq * 6 + sj
        pat_inc6 = tq * 6 + 4 + (iota & 1)

        def body(i, _):
            tok0 = tok0_w + i * _CTOK
            pltpu.sync_copy(cate_hbm.at[pl.ds(tok0 * 20 // 128,
                                              _CTOK * 20 // 128)], cate_v)
            pltpu.sync_copy(incre_hbm.at[pl.ds(tok0 * 6 // 128,
                                               _CTOK * 6 // 128)], incre_v)
            for plane in range(_PLANES):
                if plane < 5:
                    pat, src, stride = pat_cate + 4 * plane, cate_v, 80
                elif plane == 5:
                    pat, src, stride = pat_inc5, incre_v, 24
                else:
                    pat, src, stride = pat_inc6, incre_v, 24
                for v in range(crows):
                    off = pat + stride * v
                    vals = plsc.load_gather(src, [off >> 7, off & 127])
                    idx_p[plane * _GPP + v // 8,
                          pl.ds(_L * (v % 8), _L)] = vals

            for plane in range(_PLANES):
                pm2 = plane % 2
                @pl.when(jnp.logical_or(i > 0, plane >= 2))
                def _drain(pm2=pm2):
                    pltpu.make_async_copy(
                        out_hbm.at[pl.ds(0, _CTOK * 4)],
                        rows_v.at[pm2], wsem).wait()

                descs = [
                    pltpu.async_copy(
                        table_hbm.at[idx_p.at[plane * _GPP + j]],
                        rows_v.at[pm2, pl.ds(j * _GATHER_ROWS, _GATHER_ROWS)],
                        sem,
                    )
                    for j in range(_GPP)
                ]
                for d in descs:
                    d.wait()
                wdesc = pltpu.make_async_copy(
                    rows_v.at[pm2],
                    out_hbm.at[pl.ds(plane * n_tokens * 4 + tok0 * 4,
                                     _CTOK * 4)],
                    wsem)
                wdesc.start()
            return _

        lax.fori_loop(0, chunks, body, None)
        for z in range(2):
            pltpu.make_async_copy(
                out_hbm.at[pl.ds(0, _CTOK * 4)], rows_v.at[z], wsem).wait()

    return k(table, cate2d, incre2d)


def _tc_head(g3, aux, Wc, Waux, W2, b1, b2, fct, fcb):
    N = g3.shape[1]
    T = 1024
    grid = (N // T,)

    def body(g_ref, aux_ref, wc_ref, waux_ref, w2_ref, b1_ref, b2_ref,
             fct_ref, fcb_ref, rate_ref, tmu_ref, tsd_ref):
        u = jnp.dot(aux_ref[...], waux_ref[...],
                    preferred_element_type=jnp.float32)
        for k in range(_PLANES):
            u = u + jnp.dot(g_ref[k], wc_ref[pl.ds(128 * k, 128), :],
                            preferred_element_type=jnp.float32)
        h = jnp.tanh(u[:, :HID] + b1_ref[...])
        t_out = jnp.dot(h, w2_ref[...], preferred_element_type=jnp.float32)
        t_out = t_out + b2_ref[...]
        t_mu = t_out[:, :LATENT]
        tmu_ref[...] = t_mu
        tsd_ref[...] = jnp.exp(0.5 * t_out[:, LATENT:])
        v = (u[:, HID:HID + 1]
             + jnp.sum(t_mu * fct_ref[...], axis=1, keepdims=True)
             + fcb_ref[...])
        rate_ref[...] = jnp.exp(v)

    return pl.pallas_call(
        body,
        grid=grid,
        in_specs=[
            pl.BlockSpec((_PLANES, T, 128), lambda i: (0, i, 0)),
            pl.BlockSpec((T, 18), lambda i: (i, 0)),
            pl.BlockSpec((_PLANES * 128, 65), lambda i: (0, 0)),
            pl.BlockSpec((18, 65), lambda i: (0, 0)),
            pl.BlockSpec((HID, HID), lambda i: (0, 0)),
            pl.BlockSpec((1, HID), lambda i: (0, 0)),
            pl.BlockSpec((1, 2 * LATENT), lambda i: (0, 0)),
            pl.BlockSpec((1, LATENT), lambda i: (0, 0)),
            pl.BlockSpec((1, 1), lambda i: (0, 0)),
        ],
        out_specs=[
            pl.BlockSpec((T, 1), lambda i: (i, 0)),
            pl.BlockSpec((T, LATENT), lambda i: (i, 0)),
            pl.BlockSpec((T, LATENT), lambda i: (i, 0)),
        ],
        out_shape=[
            jax.ShapeDtypeStruct((N, 1), jnp.float32),
            jax.ShapeDtypeStruct((N, LATENT), jnp.float32),
            jax.ShapeDtypeStruct((N, LATENT), jnp.float32),
        ],
        compiler_params=pltpu.CompilerParams(
            dimension_semantics=("parallel",),
        ),
    )(g3, aux, Wc, Waux, W2, b1, b2, fct, fcb)


def kernel(cate, cat_incre, obs, target, time, emb_mu, emb_logvar, itemw_mu,
           t_W1, t_b1, t_W2, t_b2, fc_W, fc_b):
    B, S = cate.shape[0], cate.shape[1]
    N = B * S
    EMB = NUM_DISC * LATENT

    Wc = jnp.concatenate(
        [jnp.concatenate([t_W1[:EMB], fc_W[:EMB]], axis=1),
         jnp.zeros((_SLOTS * LATENT - EMB, HID + 1), jnp.float32)],
        axis=0)
    Waux = jnp.concatenate(
        [t_W1[EMB:EMB + NUM_CONT + 2],
         jnp.concatenate([fc_W[EMB + LATENT:], jnp.zeros((2, 1), jnp.float32)],
                         axis=0)],
        axis=1)
    fct = fc_W[EMB:EMB + LATENT].reshape(1, LATENT)

    cate2d = cate.astype(jnp.int32).reshape(N * 20 // 128, 128)
    incre2d = cat_incre.astype(jnp.int32).reshape(N * 6 // 128, 128)
    g3 = _sc_gather(emb_mu, cate2d, incre2d, N).reshape(_PLANES, N, 128)
    aux = jnp.concatenate(
        [obs.reshape(N, NUM_CONT), target.reshape(N, 1), time.reshape(N, 1)],
        axis=1)

    rate2d, t_mu, t_sd = _tc_head(
        g3, aux, Wc, Waux, t_W2,
        t_b1.reshape(1, HID), t_b2.reshape(1, 2 * LATENT),
        fct, fc_b.reshape(1, 1))

    rate = rate2d.reshape(B, S)
    t_mu = t_mu.reshape(B, S, LATENT)
    t_sd = t_sd.reshape(B, S, LATENT)

    q_item = (emb_mu, jnp.exp(0.5 * emb_logvar))
    q_itemw = (itemw_mu, jnp.ones_like(itemw_mu))
    q_time = (t_mu, t_sd)
    p_item = (jnp.zeros_like(emb_mu), jnp.ones_like(emb_mu))
    p_itemw = (jnp.zeros_like(itemw_mu), jnp.ones_like(itemw_mu))
    p_time = (jnp.zeros_like(t_mu), jnp.ones_like(t_mu))
    return (rate, q_item, q_itemw, q_time, p_item, p_itemw, p_time)
